# baseline (speedup 1.0000x reference)
import functools

import jax
import jax.numpy as jnp
from jax import lax
from jax.experimental import pallas as pl
from jax.experimental.pallas import tpu as pltpu
from jax.experimental.pallas import tpu_sc as plsc

DIM = 128
NF = 8
NCT = 16
NT = NF + NCT
FIELD = 1000
B = 4096
LANES = 16
D8 = DIM // LANES

_info = plsc.get_sparse_core_info()
NCORES = _info.num_cores
NSUB = _info.num_subcores
NW = NCORES * NSUB
BPW = B // NW

NU = 8
NSLOT = 3
NCC = 16
CCB = BPW // NCC
NCB = 3

_GDN = lax.GatherDimensionNumbers(
    offset_dims=(), collapsed_slice_dims=(0,), start_index_map=(0,))


def _splat(vec, lane):
    idx = jnp.full((LANES,), lane, dtype=jnp.int32)
    return lax.gather(vec, idx[:, None], _GDN, slice_sizes=(1,),
                      mode=lax.GatherScatterMode.PROMISE_IN_BOUNDS)


@functools.partial(
    pl.kernel,
    out_type=jax.ShapeDtypeStruct((B * NT, DIM), jnp.float32),
    mesh=plsc.VectorSubcoreMesh(core_axis_name="c", subcore_axis_name="s"),
    scratch_types=[
        pltpu.VMEM((NU, 128), jnp.float32),
        pltpu.VMEM((NU, 128), jnp.int32),
        pltpu.VMEM((NU, 128), jnp.int32),
        pltpu.VMEM((NCC, 128), jnp.int32),
        pltpu.VMEM((NSLOT * 128, DIM), jnp.float32),
        pltpu.VMEM((BPW * NCT // 128, 128), jnp.float32),
        pltpu.VMEM((BPW, NCT), jnp.float32),
        pltpu.VMEM((NCT, DIM), jnp.float32),
        pltpu.VMEM((NCB * 128, DIM), jnp.float32),
        pltpu.SemaphoreType.DMA,
        pltpu.SemaphoreType.DMA,
        pltpu.SemaphoreType.DMA,
        pltpu.SemaphoreType.DMA,
        pltpu.SemaphoreType.DMA,
        pltpu.SemaphoreType.DMA,
        pltpu.SemaphoreType.DMA,
        pltpu.SemaphoreType.DMA,
        pltpu.SemaphoreType.DMA,
        pltpu.SemaphoreType.DMA,
        pltpu.SemaphoreType.DMA,
        pltpu.SemaphoreType.DMA,
    ],
)
def _sc_embed(comb_hbm, dtab_hbm, ctab_hbm, out_hbm,
              idx_f, idx_all, dst_all, cdst_all, rows_v, act_i, act_v,
              ctab_v, cbuf_v,
              sem_i, sem_a, sem_t,
              g0, g1, g2, s0, s1, s2, c0, c1, c2):
    gsem = (g0, g1, g2)
    ssem = (s0, s1, s2)
    csem = (c0, c1, c2)
    wid = lax.axis_index("s") * NCORES + lax.axis_index("c")
    base_b = wid * BPW
    iota16 = lax.iota(jnp.int32, LANES)
    offs16 = (iota16 % NF) * FIELD

    cp_idx = pltpu.async_copy(comb_hbm.at[pl.ds(wid * NU, NU)], idx_f,
                              sem_i)
    cp_act = pltpu.async_copy(
        comb_hbm.at[pl.ds(B * NF // 128 + wid * (BPW * NCT // 128),
                          BPW * NCT // 128)], act_i, sem_a)
    cp_ct = pltpu.async_copy(ctab_hbm, ctab_v, sem_t)

    cp_idx.wait()

    def disc_prep(r, _):
        for g in range(8):
            sl = pl.ds(g * LANES, LANES)
            idx_all[r, sl] = idx_f[r, sl].astype(jnp.int32) + offs16
            k = r * 128 + g * LANES + iota16
            dst_all[r, sl] = (base_b + (k >> 3)) * NT + (k & 7)
        return 0

    lax.fori_loop(0, NU, disc_prep, 0)

    def cdst_prep(r, _):
        for g in range(8):
            sl = pl.ds(g * LANES, LANES)
            i = g * LANES + iota16
            cdst_all[r, sl] = (base_b + r * CCB + (i >> 4)) * NT + NF + (i & 15)
        return 0

    lax.fori_loop(0, NCC, cdst_prep, 0)

    def fire_g(u):
        slot = u % NSLOT
        return pltpu.async_copy(dtab_hbm.at[idx_all.at[u]],
                                rows_v.at[pl.ds(slot * 128, 128)], gsem[slot])

    def fire_s(u):
        slot = u % NSLOT
        return pltpu.async_copy(rows_v.at[pl.ds(slot * 128, 128)],
                                out_hbm.at[dst_all.at[u]], ssem[slot])

    def fire_c(cc):
        buf = cc % NCB
        return pltpu.async_copy(cbuf_v.at[pl.ds(buf * 128, 128)],
                                out_hbm.at[cdst_all.at[cc]], csem[buf])

    def compute_cc(cc):
        buf = cc % NCB

        def per_c(c, _):
            t = [ctab_v[c, pl.ds(d * LANES, LANES)] for d in range(D8)]

            def per_b2(bj, _):
                for bb in range(2):
                    b = bj * 2 + bb
                    a_v = act_v[cc * CCB + b, :]
                    s = _splat(a_v, c)
                    row = buf * 128 + b * NCT + c
                    for d in range(D8):
                        cbuf_v[row, pl.ds(d * LANES, LANES)] = t[d] * s
                return 0

            lax.fori_loop(0, CCB // 2, per_b2, 0)
            return 0

        lax.fori_loop(0, NCT, per_c, 0)

    G = [None] * NU
    S = [None] * NU
    C = [None] * NCC

    for u in range(NSLOT):
        G[u] = fire_g(u)

    cp_act.wait()
    for r in range(BPW * NCT // 128):
        for g in range(8):
            act_v[r * 8 + g, :] = act_i[r, pl.ds(g * LANES, LANES)]
    cp_ct.wait()
    compute_cc(0)

    def cont_step(cc):
        C[cc] = fire_c(cc)
        nxt = cc + 1
        if nxt < NCC:
            if nxt >= NCB:
                C[nxt - NCB].wait()
            compute_cc(nxt)

    for u in range(NU):
        if u >= NSLOT:
            S[u - NSLOT].wait()
            G[u] = fire_g(u)
        cont_step(2 * u)
        G[u].wait()
        S[u] = fire_s(u)
        cont_step(2 * u + 1)

    for u in range(NU - NSLOT, NU):
        S[u].wait()
    for cc in range(NCC - NCB, NCC):
        C[cc].wait()


def kernel(discrete_actions, continuous_actions, discrete_table,
           continuous_table):
    da = discrete_actions.astype(jnp.float32).reshape(B * NF // 128, 128)
    ca = continuous_actions.astype(jnp.float32).reshape(B * NCT // 128, 128)
    comb = jnp.concatenate([da, ca], axis=0)
    out = _sc_embed(comb, discrete_table.astype(jnp.float32),
                    continuous_table.astype(jnp.float32))
    return out.reshape(B, NT, DIM)


# speedup vs baseline: 1.0264x; 1.0264x over previous
import functools

import jax
import jax.numpy as jnp
from jax import lax
from jax.experimental import pallas as pl
from jax.experimental.pallas import tpu as pltpu
from jax.experimental.pallas import tpu_sc as plsc

DIM = 128
NF = 8
NCT = 16
NT = NF + NCT
FIELD = 1000
B = 4096
LANES = 16
D8 = DIM // LANES

_info = plsc.get_sparse_core_info()
NCORES = _info.num_cores
NSUB = _info.num_subcores
NW = NCORES * NSUB
BPW = B // NW

NU = 8
NSLOT = 3
NCC = 16
CCB = BPW // NCC
NCB = 3

_GDN = lax.GatherDimensionNumbers(
    offset_dims=(), collapsed_slice_dims=(0,), start_index_map=(0,))


def _splat(vec, lane):
    idx = jnp.full((LANES,), lane, dtype=jnp.int32)
    return lax.gather(vec, idx[:, None], _GDN, slice_sizes=(1,),
                      mode=lax.GatherScatterMode.PROMISE_IN_BOUNDS)


@functools.partial(
    pl.kernel,
    out_type=jax.ShapeDtypeStruct((B * NT, DIM), jnp.float32),
    mesh=plsc.VectorSubcoreMesh(core_axis_name="c", subcore_axis_name="s"),
    scratch_types=[
        pltpu.VMEM((NU * 8, LANES), jnp.int32),
        pltpu.VMEM((NU, 128), jnp.int32),
        pltpu.VMEM((NU, 128), jnp.int32),
        pltpu.VMEM((NCC, 128), jnp.int32),
        pltpu.VMEM((NSLOT * 128, DIM), jnp.float32),
        pltpu.VMEM((BPW, NCT), jnp.float32),
        pltpu.VMEM((NCT, DIM), jnp.float32),
        pltpu.VMEM((NCB * 128, DIM), jnp.float32),
        pltpu.SemaphoreType.DMA,
        pltpu.SemaphoreType.DMA,
        pltpu.SemaphoreType.DMA,
        pltpu.SemaphoreType.DMA,
        pltpu.SemaphoreType.DMA,
        pltpu.SemaphoreType.DMA,
        pltpu.SemaphoreType.DMA,
        pltpu.SemaphoreType.DMA,
        pltpu.SemaphoreType.DMA,
        pltpu.SemaphoreType.DMA,
        pltpu.SemaphoreType.DMA,
        pltpu.SemaphoreType.DMA,
    ],
)
def _sc_embed(didx_hbm, act_hbm, dtab_hbm, ctab_hbm, out_hbm,
              idx16, idx_all, dst_all, cdst_all, rows_v, act_v,
              ctab_v, cbuf_v,
              sem_i, sem_a, sem_t,
              g0, g1, g2, s0, s1, s2, c0, c1, c2):
    gsem = (g0, g1, g2)
    ssem = (s0, s1, s2)
    csem = (c0, c1, c2)
    wid = lax.axis_index("s") * NCORES + lax.axis_index("c")
    base_b = wid * BPW
    iota16 = lax.iota(jnp.int32, LANES)
    offs16 = (iota16 % NF) * FIELD

    cp_idx = pltpu.async_copy(didx_hbm.at[pl.ds(wid * NU * 8, NU * 8)],
                              idx16, sem_i)
    cp_act = pltpu.async_copy(act_hbm.at[pl.ds(base_b, BPW)], act_v, sem_a)
    cp_ct = pltpu.async_copy(ctab_hbm, ctab_v, sem_t)

    cp_idx.wait()

    def disc_prep(r, _):
        for g in range(8):
            sl = pl.ds(g * LANES, LANES)
            idx_all[r, sl] = idx16[r * 8 + g, :] + offs16
            k = r * 128 + g * LANES + iota16
            dst_all[r, sl] = (base_b + (k >> 3)) * NT + (k & 7)
        return 0

    lax.fori_loop(0, NU, disc_prep, 0)

    def cdst_prep(r, _):
        for g in range(8):
            sl = pl.ds(g * LANES, LANES)
            i = g * LANES + iota16
            cdst_all[r, sl] = (base_b + r * CCB + (i >> 4)) * NT + NF + (i & 15)
        return 0

    lax.fori_loop(0, NCC, cdst_prep, 0)

    def fire_g(u):
        slot = u % NSLOT
        return pltpu.async_copy(dtab_hbm.at[idx_all.at[u]],
                                rows_v.at[pl.ds(slot * 128, 128)], gsem[slot])

    def fire_s(u):
        slot = u % NSLOT
        return pltpu.async_copy(rows_v.at[pl.ds(slot * 128, 128)],
                                out_hbm.at[dst_all.at[u]], ssem[slot])

    def fire_c(cc):
        buf = cc % NCB
        return pltpu.async_copy(cbuf_v.at[pl.ds(buf * 128, 128)],
                                out_hbm.at[cdst_all.at[cc]], csem[buf])

    def compute_cc(cc):
        buf = cc % NCB

        def per_c(c, _):
            t = [ctab_v[c, pl.ds(d * LANES, LANES)] for d in range(D8)]

            def per_b2(bj, _):
                for bb in range(2):
                    b = bj * 2 + bb
                    a_v = act_v[cc * CCB + b, :]
                    s = _splat(a_v, c)
                    row = buf * 128 + b * NCT + c
                    for d in range(D8):
                        cbuf_v[row, pl.ds(d * LANES, LANES)] = t[d] * s
                return 0

            lax.fori_loop(0, CCB // 2, per_b2, 0)
            return 0

        lax.fori_loop(0, NCT, per_c, 0)

    G = [None] * NU
    S = [None] * NU
    C = [None] * NCC

    for u in range(NSLOT):
        G[u] = fire_g(u)

    cp_act.wait()
    cp_ct.wait()
    compute_cc(0)

    for u in range(NU):
        if u >= NSLOT:
            S[u - NSLOT].wait()
            G[u] = fire_g(u)
        for step in range(2):
            cc = 2 * u + step
            C[cc] = fire_c(cc)
            nxt = cc + 1
            if nxt < NCC:
                if nxt >= NCB:
                    C[nxt - NCB].wait()
                compute_cc(nxt)
        G[u].wait()
        S[u] = fire_s(u)

    for u in range(NU - NSLOT, NU):
        S[u].wait()
    for cc in range(NCC - NCB, NCC):
        C[cc].wait()


def kernel(discrete_actions, continuous_actions, discrete_table,
           continuous_table):
    da = discrete_actions.astype(jnp.int32).reshape(B * NF // LANES, LANES)
    ca = continuous_actions.astype(jnp.float32)
    out = _sc_embed(da, ca, discrete_table.astype(jnp.float32),
                    continuous_table.astype(jnp.float32))
    return out.reshape(B, NT, DIM)


# speedup vs baseline: 1.0494x; 1.0224x over previous
import functools

import jax
import jax.numpy as jnp
from jax import lax
from jax.experimental import pallas as pl
from jax.experimental.pallas import tpu as pltpu
from jax.experimental.pallas import tpu_sc as plsc

DIM = 128
NF = 8
NCT = 16
NT = NF + NCT
FIELD = 1000
B = 4096
LANES = 16
D8 = DIM // LANES

_info = plsc.get_sparse_core_info()
NCORES = _info.num_cores
NSUB = _info.num_subcores
NW = NCORES * NSUB
BPW = B // NW

NU = 8
NSLOT = 3
NCC = 16
CCB = BPW // NCC
NCB = 3

_GDN = lax.GatherDimensionNumbers(
    offset_dims=(), collapsed_slice_dims=(0,), start_index_map=(0,))


def _splat(vec, lane):
    idx = jnp.full((LANES,), lane, dtype=jnp.int32)
    return lax.gather(vec, idx[:, None], _GDN, slice_sizes=(1,),
                      mode=lax.GatherScatterMode.PROMISE_IN_BOUNDS)


@functools.partial(
    pl.kernel,
    out_type=jax.ShapeDtypeStruct((B * NT, DIM), jnp.float32),
    mesh=plsc.VectorSubcoreMesh(core_axis_name="c", subcore_axis_name="s"),
    scratch_types=[
        pltpu.VMEM((NU, 128), jnp.float32),
        pltpu.VMEM((NU, 128), jnp.int32),
        pltpu.VMEM((NU, 128), jnp.int32),
        pltpu.VMEM((NCC, 128), jnp.int32),
        pltpu.VMEM((NSLOT * 128, DIM), jnp.float32),
        pltpu.VMEM((BPW * NCT // 128, 128), jnp.float32),
        pltpu.VMEM((NCT, DIM), jnp.float32),
        pltpu.VMEM((NCB * 128, DIM), jnp.float32),
        pltpu.SemaphoreType.DMA,
        pltpu.SemaphoreType.DMA,
        pltpu.SemaphoreType.DMA,
        pltpu.SemaphoreType.DMA,
        pltpu.SemaphoreType.DMA,
        pltpu.SemaphoreType.DMA,
        pltpu.SemaphoreType.DMA,
        pltpu.SemaphoreType.DMA,
        pltpu.SemaphoreType.DMA,
        pltpu.SemaphoreType.DMA,
        pltpu.SemaphoreType.DMA,
        pltpu.SemaphoreType.DMA,
    ],
)
def _sc_embed(comb_hbm, dtab_hbm, ctab_hbm, out_hbm,
              idx_f, idx_all, dst_all, cdst_all, rows_v, act_i,
              ctab_v, cbuf_v,
              sem_i, sem_a, sem_t,
              g0, g1, g2, s0, s1, s2, c0, c1, c2):
    gsem = (g0, g1, g2)
    ssem = (s0, s1, s2)
    csem = (c0, c1, c2)
    wid = lax.axis_index("s") * NCORES + lax.axis_index("c")
    base_b = wid * BPW
    iota16 = lax.iota(jnp.int32, LANES)
    offs16 = (iota16 % NF) * FIELD

    cp_idx = pltpu.async_copy(comb_hbm.at[pl.ds(wid * NU, NU)], idx_f,
                              sem_i)
    cp_act = pltpu.async_copy(
        comb_hbm.at[pl.ds(B * NF // 128 + wid * (BPW * NCT // 128),
                          BPW * NCT // 128)], act_i, sem_a)
    cp_ct = pltpu.async_copy(ctab_hbm, ctab_v, sem_t)

    cp_idx.wait()

    def disc_prep(r, _):
        for g in range(8):
            sl = pl.ds(g * LANES, LANES)
            idx_all[r, sl] = idx_f[r, sl].astype(jnp.int32) + offs16
            k = r * 128 + g * LANES + iota16
            dst_all[r, sl] = (base_b + (k >> 3)) * NT + (k & 7)
        return 0

    lax.fori_loop(0, NU, disc_prep, 0)

    def cdst_prep(r, _):
        for g in range(8):
            sl = pl.ds(g * LANES, LANES)
            i = g * LANES + iota16
            cdst_all[r, sl] = (base_b + r * CCB + (i >> 4)) * NT + NF + (i & 15)
        return 0

    lax.fori_loop(0, NCC, cdst_prep, 0)

    def fire_g(u):
        slot = u % NSLOT
        return pltpu.async_copy(dtab_hbm.at[idx_all.at[u]],
                                rows_v.at[pl.ds(slot * 128, 128)], gsem[slot])

    def fire_s(u):
        slot = u % NSLOT
        return pltpu.async_copy(rows_v.at[pl.ds(slot * 128, 128)],
                                out_hbm.at[dst_all.at[u]], ssem[slot])

    def fire_c(cc):
        buf = cc % NCB
        return pltpu.async_copy(cbuf_v.at[pl.ds(buf * 128, 128)],
                                out_hbm.at[cdst_all.at[cc]], csem[buf])

    def compute_cc(cc):
        buf = cc % NCB

        def per_c(c, _):
            t = [ctab_v[c, pl.ds(d * LANES, LANES)] for d in range(D8)]

            def per_b2(bj, _):
                for bb in range(2):
                    b = bj * 2 + bb
                    ba = cc * CCB + b
                    a_v = act_i[ba >> 3, pl.ds((ba & 7) * LANES, LANES)]
                    s = _splat(a_v, c)
                    row = buf * 128 + b * NCT + c
                    for d in range(D8):
                        cbuf_v[row, pl.ds(d * LANES, LANES)] = t[d] * s
                return 0

            lax.fori_loop(0, CCB // 2, per_b2, 0)
            return 0

        lax.fori_loop(0, NCT, per_c, 0)

    G = [None] * NU
    S = [None] * NU
    C = [None] * NCC

    for u in range(NSLOT):
        G[u] = fire_g(u)

    cp_act.wait()
    cp_ct.wait()
    compute_cc(0)

    for u in range(NU):
        if u >= NSLOT:
            S[u - NSLOT].wait()
            G[u] = fire_g(u)
        for step in range(2):
            cc = 2 * u + step
            C[cc] = fire_c(cc)
            nxt = cc + 1
            if nxt < NCC:
                if nxt >= NCB:
                    C[nxt - NCB].wait()
                compute_cc(nxt)
        G[u].wait()
        S[u] = fire_s(u)

    for u in range(NU - NSLOT, NU):
        S[u].wait()
    for cc in range(NCC - NCB, NCC):
        C[cc].wait()


def kernel(discrete_actions, continuous_actions, discrete_table,
           continuous_table):
    da = discrete_actions.astype(jnp.float32).reshape(B * NF // 128, 128)
    ca = continuous_actions.astype(jnp.float32).reshape(B * NCT // 128, 128)
    comb = jnp.concatenate([da, ca], axis=0)
    out = _sc_embed(comb, discrete_table.astype(jnp.float32),
                    continuous_table.astype(jnp.float32))
    return out.reshape(B, NT, DIM)
